# baseline (speedup 1.0000x reference)
import jax
import jax.numpy as jnp
from jax import lax
from jax.experimental import pallas as pl
from jax.experimental.pallas import tpu as pltpu

_B, _N = 4, 4096
_TILE = 1024
_T = _N // _TILE
_PAIRS = [(i, j) for i in range(_T) for j in range(i, _T)]


def _aug(p):
    px = p[:, 0:1]
    py = p[:, 1:2]
    pz = p[:, 2:3]
    pn = (px * px + py * py) + pz * pz
    pn_hi = pn.astype(jnp.bfloat16).astype(jnp.float32)
    pn_lo = pn - pn_hi
    ones = jnp.ones_like(pn)
    return px, py, pz, pn_hi, pn_lo, ones


def _symloss_body(i_ref, j_ref, q_ref, t_ref, out_ref, acc_ref):
    b = pl.program_id(0)
    s = pl.program_id(1)
    i_blk = i_ref[s]
    j_blk = j_ref[s]

    @pl.when(s == 0)
    def _init_acc():
        acc_ref[...] = jnp.full((_T, _TILE), jnp.inf, dtype=jnp.float32)

    qx, qy, qz, qn_hi, qn_lo, ones_q = _aug(q_ref[0])
    tx, ty, tz, tn_hi, tn_lo, ones_t = _aug(t_ref[0])
    a_aug = jnp.concatenate(
        [2.0 * qx, -2.0 * qy, -2.0 * qz, qn_hi, qn_lo, ones_q, ones_q],
        axis=1)
    t_aug = jnp.concatenate(
        [tx, ty, tz, ones_t, ones_t, tn_hi, tn_lo], axis=1)
    d2 = lax.dot_general(a_aug.astype(jnp.bfloat16),
                         t_aug.astype(jnp.bfloat16),
                         (((1,), (1,)), ((), ())),
                         preferred_element_type=jnp.float32)

    rmin = jnp.min(d2, axis=1).reshape(1, _TILE)
    acc_ref[pl.ds(i_blk, 1), :] = jnp.minimum(acc_ref[pl.ds(i_blk, 1), :],
                                              rmin)

    @pl.when(i_blk != j_blk)
    def _col():
        cmin = jnp.min(d2, axis=0).reshape(1, _TILE)
        acc_ref[pl.ds(j_blk, 1), :] = jnp.minimum(
            acc_ref[pl.ds(j_blk, 1), :], cmin)

    @pl.when((b == 0) & (s == 0))
    def _init_out():
        out_ref[0, 0] = 0.0

    @pl.when(s == len(_PAIRS) - 1)
    def _flush():
        out_ref[0, 0] += jnp.sum(acc_ref[...])


def kernel(xyz):
    B, N, _ = xyz.shape
    i_idx = jnp.array([p[0] for p in _PAIRS], dtype=jnp.int32)
    j_idx = jnp.array([p[1] for p in _PAIRS], dtype=jnp.int32)
    grid_spec = pltpu.PrefetchScalarGridSpec(
        num_scalar_prefetch=2,
        grid=(B, len(_PAIRS)),
        in_specs=[
            pl.BlockSpec((1, _TILE, 3), lambda b, s, i, j: (b, i[s], 0)),
            pl.BlockSpec((1, _TILE, 3), lambda b, s, i, j: (b, j[s], 0)),
        ],
        out_specs=pl.BlockSpec((1, 1), lambda b, s, i, j: (0, 0),
                               memory_space=pltpu.SMEM),
        scratch_shapes=[pltpu.VMEM((_T, _TILE), jnp.float32)],
    )
    total = pl.pallas_call(
        _symloss_body,
        grid_spec=grid_spec,
        out_shape=jax.ShapeDtypeStruct((1, 1), jnp.float32),
    )(i_idx, j_idx, xyz, xyz)
    return total[0, 0] * (2.0 / (B * N))
